# baseline (speedup 1.0000x reference)
import functools

import jax
import jax.numpy as jnp
from jax import lax
from jax.experimental import pallas as pl
from jax.experimental.pallas import tpu as pltpu
from jax.experimental.pallas import tpu_sc as plsc

B = 4096
F = 26
K = 32
KP = 32
N = B * F
V = 1000001
NC = 2
NS = 16
NW = NC * NS
BT = 128
PER_W = F * BT
KT = K // 8
RT = V // BT
RT_MAIN = (RT // NW) * NW
PER_T = RT // NW
VPAD = (RT + 1) * BT
TAIL = V - RT * BT
TAIL_W = TAIL * KP


def _untile_sc(tab_hbm, tail_hbm, scr_hbm, slab_v, ob0_v, ob1_v,
               gsem0, gsem1, wsem0, wsem1):
    wid = lax.axis_index("s") * NC + lax.axis_index("c")
    base = wid * PER_T
    gsems = (gsem0, gsem1)
    wsems = (wsem0, wsem1)
    obs = (ob0_v, ob1_v)
    iota = lax.iota(jnp.int32, 16)

    def fire_reads(rt, q):
        for kt in range(KT):
            pltpu.async_copy(
                tab_hbm.at[pl.ds(kt * 8, 8), pl.ds(rt * BT, BT)],
                slab_v.at[q, pl.ds(kt * 8, 8), :], gsems[q])

    def wait_reads(q):
        for kt in range(KT):
            pltpu.make_async_copy(
                tab_hbm.at[pl.ds(0, 8), pl.ds(0, BT)],
                slab_v.at[q, pl.ds(kt * 8, 8), :], gsems[q]).wait()

    def fire_write(rt, q):
        pltpu.async_copy(
            obs[q], scr_hbm.at[pl.ds(rt * BT * KP, BT * KP)], wsems[q])

    def wait_write(q):
        pltpu.make_async_copy(
            obs[q], scr_hbm.at[pl.ds(0, BT * KP)], wsems[q]).wait()

    iota_kp = iota * KP

    def transpose(q):
        def rloop(rg, _):
            rvec = rg * 16 + iota
            base_vec = iota_kp + rg * (16 * KP)
            vs = [slab_v[q, c, pl.ds(rg * 16, 16)] for c in range(K)]
            for c in range(K):
                cv = (rvec + c) & (K - 1)
                plsc.store_scatter(obs[q], [base_vec + cv], vs[c])
            return _

        lax.fori_loop(0, 8, rloop, 0)

    fire_reads(base + 0, 0)
    fire_reads(base + 1, 1)
    wait_reads(0); transpose(0); fire_write(base + 0, 0); fire_reads(base + 2, 0)
    wait_reads(1); transpose(1); fire_write(base + 1, 1); fire_reads(base + 3, 1)

    def body(j, _):
        t = base + 2 * j
        wait_reads(0); wait_write(0); transpose(0); fire_write(t, 0)
        fire_reads(t + 2, 0)
        wait_reads(1); wait_write(1); transpose(1); fire_write(t + 1, 1)
        fire_reads(t + 3, 1)
        return _

    lax.fori_loop(1, PER_T // 2 - 1, body, 0)

    wait_reads(0); wait_write(0); transpose(0); fire_write(base + PER_T - 2, 0)
    wait_reads(1); wait_write(1); transpose(1); fire_write(base + PER_T - 1, 1)
    wait_write(0)
    wait_write(1)

    @pl.when(wid < RT - RT_MAIN)
    def _():
        rt = RT_MAIN + wid
        for kt in range(KT):
            pltpu.sync_copy(
                tab_hbm.at[pl.ds(kt * 8, 8), pl.ds(rt * BT, BT)],
                slab_v.at[0, pl.ds(kt * 8, 8), :])
        transpose(0)
        pltpu.sync_copy(ob0_v, scr_hbm.at[pl.ds(rt * BT * KP, BT * KP)])

    @pl.when(wid == NW - 1)
    def _():
        pltpu.sync_copy(tail_hbm, ob1_v.at[pl.ds(0, TAIL_W)])
        pltpu.sync_copy(ob1_v.at[pl.ds(0, TAIL_W)],
                        scr_hbm.at[pl.ds(RT * BT * KP, TAIL_W)])


def _gather_sc(idx_hbm, val_hbm, scr_hbm, out_hbm,
               idx_v, val_v, rows_v, ob_v,
               g0, g1, g2, g3, g4, g5, g6, g7, osem0, osem1):
    wid = lax.axis_index("s") * NC + lax.axis_index("c")
    base = wid * PER_W
    pltpu.sync_copy(idx_hbm.at[pl.ds(base, PER_W)], idx_v)
    pltpu.sync_copy(val_hbm.at[pl.ds(base, PER_W)], val_v)
    gsems = (g0, g1, g2, g3, g4, g5, g6, g7)
    osems = (osem0, osem1)
    iota = lax.iota(jnp.int32, 16)
    SLOTS = 8

    def fire(f):
        s = f % SLOTS
        return pltpu.async_copy(
            scr_hbm.at[idx_v.at[pl.ds(f * BT, BT)]],
            rows_v.at[pl.ds(s * BT, BT)], gsems[s])

    gathers = [fire(f) for f in range(SLOTS)]
    pending = [None, None]
    for f in range(F):
        s = f % SLOTS
        p = f & 1
        gathers[s].wait()
        if pending[p] is not None:
            for c in pending[p]:
                c.wait()

        def rloop(j, _):
            rb = s * BT + j
            lane = jnp.full((16,), j & 15, jnp.int32)
            vals = val_v[pl.ds(f * BT + (j & 0x70), 16)]
            vj = vals.at[lane].get(mode="promise_in_bounds")
            ridx = idx_v[pl.ds(f * BT + (j & 0x70), 16)]
            rj = ridx.at[lane].get(mode="promise_in_bounds")
            rowsplat = jnp.full((16,), rb, jnp.int32)
            lo_c = (iota + rj) & (K - 1)
            hi_c = (iota + 16 + rj) & (K - 1)
            bvec = jnp.full((16,), j, jnp.int32)
            lo = plsc.load_gather(rows_v, [rowsplat, lo_c]) * vj
            hi = plsc.load_gather(rows_v, [rowsplat, hi_c]) * vj
            plsc.store_scatter(ob_v.at[p], [iota, bvec], lo)
            plsc.store_scatter(ob_v.at[p], [iota + 16, bvec], hi)
            return _

        lax.fori_loop(0, BT, rloop, 0, unroll=2)

        cps = []
        for kt in range(KT):
            cps.append(pltpu.async_copy(
                ob_v.at[p, pl.ds(kt * 8, 8), pl.ds(0, BT)],
                out_hbm.at[f, kt, wid], osems[p]))
        pending[p] = cps
        if f + SLOTS < F:
            gathers[s] = fire(f + SLOTS)
    for pend in pending:
        if pend is not None:
            for c in pend:
                c.wait()


@jax.jit
def _fm(idx_flat, val_flat, table_t, tail_flat):
    mesh = plsc.VectorSubcoreMesh(core_axis_name="c", subcore_axis_name="s")

    untile = functools.partial(
        pl.kernel,
        mesh=mesh,
        out_type=jax.ShapeDtypeStruct((VPAD * KP,), jnp.float32),
        scratch_types=[
            pltpu.VMEM((2, K, BT), jnp.float32),
            pltpu.VMEM((BT * KP,), jnp.float32),
            pltpu.VMEM((BT * KP,), jnp.float32),
            pltpu.SemaphoreType.DMA,
            pltpu.SemaphoreType.DMA,
            pltpu.SemaphoreType.DMA,
            pltpu.SemaphoreType.DMA,
        ],
        compiler_params=pltpu.CompilerParams(
            use_tc_tiling_on_sc=True, needs_layout_passes=False),
    )(_untile_sc)
    scratch = untile(table_t, tail_flat).reshape(VPAD, KP)

    gather = functools.partial(
        pl.kernel,
        mesh=mesh,
        out_type=jax.ShapeDtypeStruct((F, KT, NW, 8, BT), jnp.float32),
        scratch_types=[
            pltpu.VMEM((PER_W,), jnp.int32),
            pltpu.VMEM((PER_W,), jnp.float32),
            pltpu.VMEM((8 * BT, KP), jnp.float32),
            pltpu.VMEM((2, K, 129), jnp.float32),
        ] + [pltpu.SemaphoreType.DMA] * 10,
        compiler_params=pltpu.CompilerParams(
            use_tc_tiling_on_sc=False, needs_layout_passes=False),
    )(_gather_sc)
    return gather(idx_flat, val_flat, scratch)


def kernel(nonzero_index, nonzero_value, table):
    def to_blocks(x):
        return (x.reshape(NW, BT, F).transpose(0, 2, 1).reshape(N))

    idx_flat = to_blocks(nonzero_index.astype(jnp.int32))
    val_flat = to_blocks(nonzero_value)
    tail = table[RT * BT:, :]
    cols = (jnp.arange(K)[None, :] - jnp.arange(TAIL)[:, None]) % K
    tail_rot = jnp.take_along_axis(tail, cols, axis=1)
    tail_flat = tail_rot.reshape(TAIL * KP)
    o5 = _fm(idx_flat, val_flat, table.T, tail_flat)
    o = o5.transpose(2, 4, 0, 1, 3)
    return o.reshape(B, F, K)


# speedup vs baseline: 1.1643x; 1.1643x over previous
import functools

import jax
import jax.numpy as jnp
from jax import lax
from jax.experimental import pallas as pl
from jax.experimental.pallas import tpu as pltpu
from jax.experimental.pallas import tpu_sc as plsc

B = 4096
F = 26
K = 32
KP = 40
N = B * F
V = 1000001
NC = 2
NS = 16
NW = NC * NS
BT = 128
PER_W = F * BT
KT = K // 8
RT = V // BT
RT_MAIN = (RT // NW) * NW
PER_T = RT // NW
VPAD = (RT + 1) * BT
TAIL = V - RT * BT
TAIL_W = TAIL * KP


def _untile_sc(tab_hbm, tail_hbm, scr_hbm, slab_v, ob0_v, ob1_v,
               gsem0, gsem1, wsem0, wsem1):
    wid = lax.axis_index("s") * NC + lax.axis_index("c")
    base = wid * PER_T
    gsems = (gsem0, gsem1)
    wsems = (wsem0, wsem1)
    obs = (ob0_v, ob1_v)
    iota = lax.iota(jnp.int32, 16)

    def fire_reads(rt, q):
        for kt in range(KT):
            pltpu.async_copy(
                tab_hbm.at[pl.ds(kt * 8, 8), pl.ds(rt * BT, BT)],
                slab_v.at[q, pl.ds(kt * 8, 8), :], gsems[q])

    def wait_reads(q):
        for kt in range(KT):
            pltpu.make_async_copy(
                tab_hbm.at[pl.ds(0, 8), pl.ds(0, BT)],
                slab_v.at[q, pl.ds(kt * 8, 8), :], gsems[q]).wait()

    def fire_write(rt, q):
        pltpu.async_copy(
            obs[q], scr_hbm.at[pl.ds(rt * BT * KP, BT * KP)], wsems[q])

    def wait_write(q):
        pltpu.make_async_copy(
            obs[q], scr_hbm.at[pl.ds(0, BT * KP)], wsems[q]).wait()

    iota_kp = iota * KP

    def transpose(q):
        def rloop(rg, _):
            rbase = rg * (16 * KP)
            vs = [slab_v[q, c, pl.ds(rg * 16, 16)] for c in range(K)]
            for c in range(K):
                plsc.store_scatter(obs[q], [iota_kp + (rbase + c)], vs[c])
            return _

        lax.fori_loop(0, 8, rloop, 0)

    fire_reads(base + 0, 0)
    fire_reads(base + 1, 1)
    wait_reads(0); transpose(0); fire_write(base + 0, 0); fire_reads(base + 2, 0)
    wait_reads(1); transpose(1); fire_write(base + 1, 1); fire_reads(base + 3, 1)

    def body(j, _):
        t = base + 2 * j
        wait_reads(0); wait_write(0); transpose(0); fire_write(t, 0)
        fire_reads(t + 2, 0)
        wait_reads(1); wait_write(1); transpose(1); fire_write(t + 1, 1)
        fire_reads(t + 3, 1)
        return _

    lax.fori_loop(1, PER_T // 2 - 1, body, 0)

    wait_reads(0); wait_write(0); transpose(0); fire_write(base + PER_T - 2, 0)
    wait_reads(1); wait_write(1); transpose(1); fire_write(base + PER_T - 1, 1)
    wait_write(0)
    wait_write(1)

    @pl.when(wid < RT - RT_MAIN)
    def _():
        rt = RT_MAIN + wid
        for kt in range(KT):
            pltpu.sync_copy(
                tab_hbm.at[pl.ds(kt * 8, 8), pl.ds(rt * BT, BT)],
                slab_v.at[0, pl.ds(kt * 8, 8), :])
        transpose(0)
        pltpu.sync_copy(ob0_v, scr_hbm.at[pl.ds(rt * BT * KP, BT * KP)])

    @pl.when(wid == NW - 1)
    def _():
        pltpu.sync_copy(tail_hbm, ob1_v.at[pl.ds(0, TAIL_W)])
        pltpu.sync_copy(ob1_v.at[pl.ds(0, TAIL_W)],
                        scr_hbm.at[pl.ds(RT * BT * KP, TAIL_W)])


def _gather_sc(idx_hbm, val_hbm, scr_hbm, out_hbm,
               idx_v, val_v, rows_v, ob_v,
               g0, g1, g2, g3, g4, g5, g6, g7, osem0, osem1):
    wid = lax.axis_index("s") * NC + lax.axis_index("c")
    base = wid * PER_W
    pltpu.sync_copy(idx_hbm.at[pl.ds(base, PER_W)], idx_v)
    pltpu.sync_copy(val_hbm.at[pl.ds(base, PER_W)], val_v)
    gsems = (g0, g1, g2, g3, g4, g5, g6, g7)
    osems = (osem0, osem1)
    iota = lax.iota(jnp.int32, 16)
    SLOTS = 8

    def fire(f):
        s = f % SLOTS
        return pltpu.async_copy(
            scr_hbm.at[idx_v.at[pl.ds(f * BT, BT)]],
            rows_v.at[pl.ds(s * BT, BT)], gsems[s])

    gathers = [fire(f) for f in range(SLOTS)]
    pending = [None, None]
    for f in range(F):
        s = f % SLOTS
        p = f & 1
        gathers[s].wait()
        if pending[p] is not None:
            for c in pending[p]:
                c.wait()

        def rloop(rg, _):
            g16 = rg * 16
            vals = val_v[pl.ds(f * BT + g16, 16)]
            rows = [(rows_v[s * BT + g16 + jj, pl.ds(0, 16)],
                     rows_v[s * BT + g16 + jj, pl.ds(16, 16)])
                    for jj in range(16)]
            for jj in range(16):
                vj = vals.at[jnp.full((16,), jj, jnp.int32)].get(
                    mode="promise_in_bounds")
                bvec = g16 + jj
                bsplat = jnp.full((16,), bvec, jnp.int32)
                plsc.store_scatter(ob_v.at[p], [iota, bsplat], rows[jj][0] * vj)
                plsc.store_scatter(ob_v.at[p], [iota + 16, bsplat],
                                   rows[jj][1] * vj)
            return _

        lax.fori_loop(0, 8, rloop, 0)

        cps = []
        for kt in range(KT):
            cps.append(pltpu.async_copy(
                ob_v.at[p, pl.ds(kt * 8, 8), pl.ds(0, BT)],
                out_hbm.at[f, kt, wid], osems[p]))
        pending[p] = cps
        if f + SLOTS < F:
            gathers[s] = fire(f + SLOTS)
    for pend in pending:
        if pend is not None:
            for c in pend:
                c.wait()


@jax.jit
def _fm(idx_flat, val_flat, table_t, tail_flat):
    mesh = plsc.VectorSubcoreMesh(core_axis_name="c", subcore_axis_name="s")

    untile = functools.partial(
        pl.kernel,
        mesh=mesh,
        out_type=jax.ShapeDtypeStruct((VPAD * KP,), jnp.float32),
        scratch_types=[
            pltpu.VMEM((2, K, BT), jnp.float32),
            pltpu.VMEM((BT * KP,), jnp.float32),
            pltpu.VMEM((BT * KP,), jnp.float32),
            pltpu.SemaphoreType.DMA,
            pltpu.SemaphoreType.DMA,
            pltpu.SemaphoreType.DMA,
            pltpu.SemaphoreType.DMA,
        ],
        compiler_params=pltpu.CompilerParams(
            use_tc_tiling_on_sc=True, needs_layout_passes=False),
    )(_untile_sc)
    scratch = untile(table_t, tail_flat).reshape(VPAD, KP)

    gather = functools.partial(
        pl.kernel,
        mesh=mesh,
        out_type=jax.ShapeDtypeStruct((F, KT, NW, 8, BT), jnp.float32),
        scratch_types=[
            pltpu.VMEM((PER_W,), jnp.int32),
            pltpu.VMEM((PER_W,), jnp.float32),
            pltpu.VMEM((8 * BT, KP), jnp.float32),
            pltpu.VMEM((2, K, 129), jnp.float32),
        ] + [pltpu.SemaphoreType.DMA] * 10,
        compiler_params=pltpu.CompilerParams(
            use_tc_tiling_on_sc=False, needs_layout_passes=False),
    )(_gather_sc)
    return gather(idx_flat, val_flat, scratch)


def kernel(nonzero_index, nonzero_value, table):
    def to_blocks(x):
        return (x.reshape(NW, BT, F).transpose(0, 2, 1).reshape(N))

    idx_flat = to_blocks(nonzero_index.astype(jnp.int32))
    val_flat = to_blocks(nonzero_value)
    tail = jnp.pad(table[RT * BT:, :], ((0, 0), (0, KP - K)))
    tail_flat = tail.reshape(TAIL * KP)
    o5 = _fm(idx_flat, val_flat, table.T, tail_flat)
    o = o5.transpose(2, 4, 0, 1, 3)
    return o.reshape(B, F, K)
